# baseline (speedup 1.0000x reference)
import functools

import jax
import jax.numpy as jnp
import numpy as np
from jax.experimental import pallas as pl

_NUM_SAMPLES = 128
_NEAR = 0.1
_FAR = 100.0
_BN = 1024


def _expand_matrix():
    e = np.zeros((3, 3 * _NUM_SAMPLES), dtype=np.float32)
    for s in range(_NUM_SAMPLES):
        for c in range(3):
            e[c, 3 * s + c] = 1.0
    return jnp.asarray(e)


def _body(o_ref, d_ref, e_ref, pos_ref, view_ref, dep_ref, del_ref):
    o = o_ref[:]
    d = d_ref[:]
    e = e_ref[:]
    dn = d / (jnp.sqrt(jnp.sum(d * d, axis=1, keepdims=True)) + 1e-8)
    view = jax.lax.dot_general(
        dn, e, (((1,), (0,)), ((), ())),
        preferred_element_type=jnp.float32,
        precision=jax.lax.Precision.HIGHEST,
    )
    obrd = jax.lax.dot_general(
        o, e, (((1,), (0,)), ((), ())),
        preferred_element_type=jnp.float32,
        precision=jax.lax.Precision.HIGHEST,
    )
    step = (_FAR - _NEAR) / (_NUM_SAMPLES - 1)
    lane = jax.lax.broadcasted_iota(jnp.int32, (1, 3 * _NUM_SAMPLES), 1)
    depth_row = _NEAR + (lane // 3).astype(jnp.float32) * step
    pos_ref[:] = obrd + view * depth_row
    view_ref[:] = view
    s_lane = jax.lax.broadcasted_iota(jnp.int32, (_BN, _NUM_SAMPLES), 1)
    dep_ref[:] = _NEAR + s_lane.astype(jnp.float32) * step
    del_ref[:] = jnp.full((_BN, _NUM_SAMPLES), step, dtype=jnp.float32)


@jax.jit
def kernel(origins, directions):
    n, _ = origins.shape
    s = _NUM_SAMPLES
    grid = (n // _BN,)
    pos2d, view2d, depths, deltas = pl.pallas_call(
        _body,
        grid=grid,
        in_specs=[
            pl.BlockSpec((_BN, 3), lambda i: (i, 0)),
            pl.BlockSpec((_BN, 3), lambda i: (i, 0)),
            pl.BlockSpec((3, 3 * s), lambda i: (0, 0)),
        ],
        out_specs=[
            pl.BlockSpec((_BN, 3 * s), lambda i: (i, 0)),
            pl.BlockSpec((_BN, 3 * s), lambda i: (i, 0)),
            pl.BlockSpec((_BN, s), lambda i: (i, 0)),
            pl.BlockSpec((_BN, s), lambda i: (i, 0)),
        ],
        out_shape=[
            jax.ShapeDtypeStruct((n, 3 * s), jnp.float32),
            jax.ShapeDtypeStruct((n, 3 * s), jnp.float32),
            jax.ShapeDtypeStruct((n, s), jnp.float32),
            jax.ShapeDtypeStruct((n, s), jnp.float32),
        ],
    )(origins, directions, _expand_matrix())
    positions = pos2d.reshape(n, s, 3)
    view_directions = view2d.reshape(n, s, 3)
    return positions, view_directions, depths, deltas


# speedup vs baseline: 5.1431x; 5.1431x over previous
import jax
import jax.numpy as jnp
from jax.experimental import pallas as pl

_NUM_SAMPLES = 128
_NEAR = 0.1
_FAR = 100.0
_BN = 1024


def _body(o_ref, d_ref, pos_ref, view_ref, dep_ref, del_ref):
    o = o_ref[:]
    d = d_ref[:]
    dn = d / (jnp.sqrt(jnp.sum(d * d, axis=1, keepdims=True)) + 1e-8)
    step = (_FAR - _NEAR) / (_NUM_SAMPLES - 1)
    lane = jax.lax.broadcasted_iota(jnp.int32, (1, _NUM_SAMPLES), 1)
    depth_row = _NEAR + lane.astype(jnp.float32) * step
    for c in range(3):
        oc = o[:, c : c + 1]
        dc = dn[:, c : c + 1]
        pos_ref[c] = oc + dc * depth_row
        view_ref[c] = jnp.broadcast_to(dc, (_BN, _NUM_SAMPLES))
    dep_ref[:] = jnp.broadcast_to(depth_row, (_BN, _NUM_SAMPLES))
    del_ref[:] = jnp.full((_BN, _NUM_SAMPLES), step, dtype=jnp.float32)


@jax.jit
def kernel(origins, directions):
    n, _ = origins.shape
    s = _NUM_SAMPLES
    grid = (n // _BN,)
    pos3, view3, depths, deltas = pl.pallas_call(
        _body,
        grid=grid,
        in_specs=[
            pl.BlockSpec((_BN, 3), lambda i: (i, 0)),
            pl.BlockSpec((_BN, 3), lambda i: (i, 0)),
        ],
        out_specs=[
            pl.BlockSpec((3, _BN, s), lambda i: (0, i, 0)),
            pl.BlockSpec((3, _BN, s), lambda i: (0, i, 0)),
            pl.BlockSpec((_BN, s), lambda i: (i, 0)),
            pl.BlockSpec((_BN, s), lambda i: (i, 0)),
        ],
        out_shape=[
            jax.ShapeDtypeStruct((3, n, s), jnp.float32),
            jax.ShapeDtypeStruct((3, n, s), jnp.float32),
            jax.ShapeDtypeStruct((n, s), jnp.float32),
            jax.ShapeDtypeStruct((n, s), jnp.float32),
        ],
    )(origins, directions)
    positions = pos3.transpose(1, 2, 0)
    view_directions = view3.transpose(1, 2, 0)
    return positions, view_directions, depths, deltas


# speedup vs baseline: 7.2767x; 1.4148x over previous
import jax
import jax.numpy as jnp
from jax.experimental import pallas as pl

_NUM_SAMPLES = 128
_NEAR = 0.1
_FAR = 100.0
_BN = 1024


def _body(o_ref, d_ref, pos_ref, view_ref, dep_ref, del_ref):
    o = jnp.transpose(o_ref[:])
    d = jnp.transpose(d_ref[:])
    dn = d / (jnp.sqrt(jnp.sum(d * d, axis=1, keepdims=True)) + 1e-8)
    step = (_FAR - _NEAR) / (_NUM_SAMPLES - 1)
    lane = jax.lax.broadcasted_iota(jnp.int32, (1, _NUM_SAMPLES), 1)
    depth_row = _NEAR + lane.astype(jnp.float32) * step
    for c in range(3):
        oc = o[:, c : c + 1]
        dc = dn[:, c : c + 1]
        pos_ref[c] = oc + dc * depth_row
        view_ref[c] = jnp.broadcast_to(dc, (_BN, _NUM_SAMPLES))
    dep_ref[:] = jnp.broadcast_to(depth_row, (_BN, _NUM_SAMPLES))
    del_ref[:] = jnp.full((_BN, _NUM_SAMPLES), step, dtype=jnp.float32)


@jax.jit
def kernel(origins, directions):
    n, _ = origins.shape
    s = _NUM_SAMPLES
    grid = (n // _BN,)
    pos3, view3, depths, deltas = pl.pallas_call(
        _body,
        grid=grid,
        in_specs=[
            pl.BlockSpec((3, _BN), lambda i: (0, i)),
            pl.BlockSpec((3, _BN), lambda i: (0, i)),
        ],
        out_specs=[
            pl.BlockSpec((3, _BN, s), lambda i: (0, i, 0)),
            pl.BlockSpec((3, _BN, s), lambda i: (0, i, 0)),
            pl.BlockSpec((_BN, s), lambda i: (i, 0)),
            pl.BlockSpec((_BN, s), lambda i: (i, 0)),
        ],
        out_shape=[
            jax.ShapeDtypeStruct((3, n, s), jnp.float32),
            jax.ShapeDtypeStruct((3, n, s), jnp.float32),
            jax.ShapeDtypeStruct((n, s), jnp.float32),
            jax.ShapeDtypeStruct((n, s), jnp.float32),
        ],
    )(origins.T, directions.T)
    positions = pos3.transpose(1, 2, 0)
    view_directions = view3.transpose(1, 2, 0)
    return positions, view_directions, depths, deltas


# speedup vs baseline: 8.1467x; 1.1196x over previous
import jax
import jax.numpy as jnp
from jax.experimental import pallas as pl

_NUM_SAMPLES = 128
_NEAR = 0.1
_FAR = 100.0
_BN = 2048


def _body(o_ref, d_ref, pos_ref, view_ref, dep_ref, del_ref):
    o = jnp.transpose(o_ref[:])
    d = jnp.transpose(d_ref[:])
    dn = d / (jnp.sqrt(jnp.sum(d * d, axis=1, keepdims=True)) + 1e-8)
    step = (_FAR - _NEAR) / (_NUM_SAMPLES - 1)
    lane = jax.lax.broadcasted_iota(jnp.int32, (1, _NUM_SAMPLES), 1)
    depth_row = _NEAR + lane.astype(jnp.float32) * step
    for c in range(3):
        oc = o[:, c : c + 1]
        dc = dn[:, c : c + 1]
        pos_ref[c] = oc + dc * depth_row
        view_ref[c] = jnp.broadcast_to(dc, (_BN, _NUM_SAMPLES))
    dep_ref[:] = jnp.broadcast_to(depth_row, (_BN, _NUM_SAMPLES))
    del_ref[:] = jnp.full((_BN, _NUM_SAMPLES), step, dtype=jnp.float32)


@jax.jit
def kernel(origins, directions):
    n, _ = origins.shape
    s = _NUM_SAMPLES
    grid = (n // _BN,)
    pos3, view3, depths, deltas = pl.pallas_call(
        _body,
        grid=grid,
        in_specs=[
            pl.BlockSpec((3, _BN), lambda i: (0, i)),
            pl.BlockSpec((3, _BN), lambda i: (0, i)),
        ],
        out_specs=[
            pl.BlockSpec((3, _BN, s), lambda i: (0, i, 0)),
            pl.BlockSpec((3, _BN, s), lambda i: (0, i, 0)),
            pl.BlockSpec((_BN, s), lambda i: (i, 0)),
            pl.BlockSpec((_BN, s), lambda i: (i, 0)),
        ],
        out_shape=[
            jax.ShapeDtypeStruct((3, n, s), jnp.float32),
            jax.ShapeDtypeStruct((3, n, s), jnp.float32),
            jax.ShapeDtypeStruct((n, s), jnp.float32),
            jax.ShapeDtypeStruct((n, s), jnp.float32),
        ],
    )(origins.T, directions.T)
    positions = pos3.transpose(1, 2, 0)
    view_directions = view3.transpose(1, 2, 0)
    return positions, view_directions, depths, deltas


# speedup vs baseline: 8.4333x; 1.0352x over previous
import jax
import jax.numpy as jnp
from jax.experimental import pallas as pl

_NUM_SAMPLES = 128
_NEAR = 0.1
_FAR = 100.0
_BN = 4096


def _body(o_ref, d_ref, pos_ref, view_ref, dep_ref, del_ref):
    o = jnp.transpose(o_ref[:])
    d = jnp.transpose(d_ref[:])
    dn = d / (jnp.sqrt(jnp.sum(d * d, axis=1, keepdims=True)) + 1e-8)
    step = (_FAR - _NEAR) / (_NUM_SAMPLES - 1)
    lane = jax.lax.broadcasted_iota(jnp.int32, (1, _NUM_SAMPLES), 1)
    depth_row = _NEAR + lane.astype(jnp.float32) * step
    for c in range(3):
        oc = o[:, c : c + 1]
        dc = dn[:, c : c + 1]
        pos_ref[c] = oc + dc * depth_row
        view_ref[c] = jnp.broadcast_to(dc, (_BN, _NUM_SAMPLES))
    dep_ref[:] = jnp.broadcast_to(depth_row, (_BN, _NUM_SAMPLES))
    del_ref[:] = jnp.full((_BN, _NUM_SAMPLES), step, dtype=jnp.float32)


@jax.jit
def kernel(origins, directions):
    n, _ = origins.shape
    s = _NUM_SAMPLES
    grid = (n // _BN,)
    pos3, view3, depths, deltas = pl.pallas_call(
        _body,
        grid=grid,
        in_specs=[
            pl.BlockSpec((3, _BN), lambda i: (0, i)),
            pl.BlockSpec((3, _BN), lambda i: (0, i)),
        ],
        out_specs=[
            pl.BlockSpec((3, _BN, s), lambda i: (0, i, 0)),
            pl.BlockSpec((3, _BN, s), lambda i: (0, i, 0)),
            pl.BlockSpec((_BN, s), lambda i: (i, 0)),
            pl.BlockSpec((_BN, s), lambda i: (i, 0)),
        ],
        out_shape=[
            jax.ShapeDtypeStruct((3, n, s), jnp.float32),
            jax.ShapeDtypeStruct((3, n, s), jnp.float32),
            jax.ShapeDtypeStruct((n, s), jnp.float32),
            jax.ShapeDtypeStruct((n, s), jnp.float32),
        ],
    )(origins.T, directions.T)
    positions = pos3.transpose(1, 2, 0)
    view_directions = view3.transpose(1, 2, 0)
    return positions, view_directions, depths, deltas
